# baseline (speedup 1.0000x reference)
import math

import jax
import jax.numpy as jnp
from jax import lax
from jax.experimental import pallas as pl
from jax.experimental.pallas import tpu as pltpu
from jax.experimental.pallas import tpu_sc as plsc

_ABSTAIN_WEIGHT = 0.1
_NUM_WORKERS = 32
_LANES = 16


def _sc_body(nc, ns, piece, npiece,
             clp_hbm, clpt_hbm, idxt_hbm, s2c_hbm,
             partmax_hbm, lpc_hbm,
             lmax_v, val_v, idx_v, gidx_v, gout_v,
             vals_s, idxs_s,
             sem0, sem1, sem):
    sub = piece // _LANES
    spw = ns // _NUM_WORKERS
    cid = lax.axis_index("c")
    sid = lax.axis_index("s")
    wid = cid * _LANES + sid

    lane = lax.iota(jnp.int32, _LANES)
    ninf = jnp.full((_LANES,), -jnp.inf, jnp.float32)

    sems = (sem0, sem1)
    cpiece = piece * _LANES
    chunk = piece * npiece

    def _hbm_slice(p):
        return pl.ds(cid * _LANES * chunk + p * cpiece + sid * piece, piece)

    def _spmem_slice(b):
        return pl.ds(b * cpiece + sid * piece, piece)

    def _issue(p):
        b = p % 2
        pltpu.async_copy(clpt_hbm.at[_hbm_slice(p)], vals_s.at[_spmem_slice(b)],
                         sems[b])
        pltpu.async_copy(idxt_hbm.at[_hbm_slice(p)], idxs_s.at[_spmem_slice(b)],
                         sems[b])

    def _wait(p):
        b = p % 2
        pltpu.make_async_copy(clpt_hbm.at[_hbm_slice(p)],
                              vals_s.at[_spmem_slice(b)], sems[b]).wait()
        pltpu.make_async_copy(idxt_hbm.at[_hbm_slice(p)],
                              idxs_s.at[_spmem_slice(b)], sems[b]).wait()

    _issue(0)
    if npiece > 1:
        _issue(1)

    @plsc.parallel_loop(0, (ns + _LANES) // _LANES, unroll=8)
    def _init(i):
        lmax_v[pl.ds(i * _LANES, _LANES)] = ninf

    priv = ns + lane

    for p in range(npiece):
        b = p % 2
        _wait(p)
        pltpu.sync_copy(vals_s.at[_spmem_slice(b)], val_v)
        pltpu.sync_copy(idxs_s.at[_spmem_slice(b)], idx_v)
        if p + 2 < npiece:
            _issue(p + 2)

        hs = idx_v[pl.ds(0, _LANES)]

        def _rake(t, carry):
            m, cs, tgt = carry
            row = t * _LANES
            v = val_v[pl.ds(row, _LANES)]
            s = idx_v[pl.ds(row, _LANES)]
            changed = s != cs
            plsc.store_scatter(lmax_v, [tgt], m, mask=changed)
            tgt = jnp.where(changed, s, tgt)
            m = jnp.where(changed, v, jnp.maximum(m, v))
            return m, s, tgt

        m, cs, _ = plsc.parallel_loop(
            0, sub, unroll=8, carry=(ninf, hs, priv))(_rake)

        headv = plsc.load_gather(lmax_v, [priv])
        for j in range(_LANES):
            g = plsc.load_gather(lmax_v, [hs])
            plsc.store_scatter(lmax_v, [hs], jnp.maximum(g, headv),
                               mask=lane == j)
        for j in range(_LANES):
            g = plsc.load_gather(lmax_v, [cs])
            plsc.store_scatter(lmax_v, [cs], jnp.maximum(g, m),
                               mask=lane == j)
        lmax_v[pl.ds(ns, _LANES)] = ninf

    pltpu.sync_copy(lmax_v.at[pl.ds(0, ns)], vals_s.at[pl.ds(sid * ns, ns)])
    out_copy = pltpu.async_copy(vals_s.at[pl.ds(sid * ns, ns)],
                                partmax_hbm.at[pl.ds(wid * ns, ns)], sem0)

    sbase = wid * spw
    pltpu.sync_copy(s2c_hbm.at[pl.ds(sbase, spw)], gidx_v)
    pltpu.async_copy(clp_hbm.at[gidx_v], gout_v, sem).wait()
    pltpu.sync_copy(gout_v, lpc_hbm.at[pl.ds(sbase, spw)])

    out_copy.wait()


def _tc_body(pm_ref, tail_ref, lpc_ref, bug_ref, np_ref,
             loss_ref, nnp_ref, corr_ref, nb_ref, nbc_ref):
    pm = pm_ref[...]
    tail = tail_ref[...]
    lpc = lpc_ref[...]
    bug = bug_ref[...]
    nonpad = np_ref[...]

    is_bug = bug == 1
    is_np = nonpad == 1

    seg_max = jnp.maximum(jnp.max(pm, axis=0), tail)
    lp = jnp.where(is_bug, lpc, tail)
    lp = jnp.where(is_np, lp, 0.0)
    lp = jnp.minimum(lp, math.log(0.995))
    lp = lp + jnp.where(jnp.logical_and(is_bug, is_np),
                        _ABSTAIN_WEIGHT * tail,
                        jnp.zeros_like(lp))

    nnp = jnp.sum(nonpad)
    correct = jnp.logical_and(seg_max == lp, is_np)
    nobug = jnp.logical_and(jnp.logical_not(is_bug), is_np)

    loss_ref[0, 0] = -jnp.sum(lp) / nnp.astype(jnp.float32)
    nnp_ref[0, 0] = nnp
    corr_ref[0, 0] = jnp.sum(correct.astype(jnp.int32))
    nb_ref[0, 0] = jnp.sum(nobug.astype(jnp.int32))
    nbc_ref[0, 0] = jnp.sum(jnp.logical_and(nobug, correct).astype(jnp.int32))


def kernel(candidate_log_probs, candidate_to_sample_idx, sample_has_bug,
           sample_to_correct_candidate_idx, sample_is_nonpad, train_step):
    nc = candidate_to_sample_idx.shape[0]
    ns = sample_has_bug.shape[0]
    npiece = 4
    piece = nc // (_NUM_WORKERS * npiece)
    sub = piece // _LANES

    clp_t = (candidate_log_probs[:nc]
             .reshape(-1, _LANES, sub).swapaxes(1, 2).reshape(-1))
    idx_t = (candidate_to_sample_idx
             .reshape(-1, _LANES, sub).swapaxes(1, 2).reshape(-1))

    sc_fn = pl.kernel(
        lambda *refs: _sc_body(nc, ns, piece, npiece, *refs),
        out_type=(
            jax.ShapeDtypeStruct((_NUM_WORKERS * ns,), jnp.float32),
            jax.ShapeDtypeStruct((ns,), jnp.float32),
        ),
        mesh=plsc.VectorSubcoreMesh(core_axis_name="c", subcore_axis_name="s"),
        compiler_params=pltpu.CompilerParams(needs_layout_passes=False),
        scratch_types=[
            pltpu.VMEM((ns + _LANES,), jnp.float32),
            pltpu.VMEM((piece,), jnp.float32),
            pltpu.VMEM((piece,), jnp.int32),
            pltpu.VMEM((ns // _NUM_WORKERS,), jnp.int32),
            pltpu.VMEM((ns // _NUM_WORKERS,), jnp.float32),
            pltpu.VMEM_SHARED((2 * piece * _LANES,), jnp.float32),
            pltpu.VMEM_SHARED((2 * piece * _LANES,), jnp.int32),
            pltpu.SemaphoreType.DMA,
            pltpu.SemaphoreType.DMA,
            pltpu.SemaphoreType.DMA,
        ],
    )
    partmax, lpc = sc_fn(candidate_log_probs, clp_t, idx_t,
                         sample_to_correct_candidate_idx)

    srows = ns // 128
    outs = pl.pallas_call(
        _tc_body,
        out_shape=(
            jax.ShapeDtypeStruct((1, 1), jnp.float32),
            jax.ShapeDtypeStruct((1, 1), jnp.int32),
            jax.ShapeDtypeStruct((1, 1), jnp.int32),
            jax.ShapeDtypeStruct((1, 1), jnp.int32),
            jax.ShapeDtypeStruct((1, 1), jnp.int32),
        ),
        out_specs=tuple(pl.BlockSpec(memory_space=pltpu.SMEM)
                        for _ in range(5)),
    )(
        partmax.reshape(_NUM_WORKERS, srows, 128),
        candidate_log_probs[nc:].reshape(srows, 128),
        lpc.reshape(srows, 128),
        sample_has_bug.astype(jnp.int32).reshape(srows, 128),
        sample_is_nonpad.astype(jnp.int32).reshape(srows, 128),
    )
    loss, nnp, corr, nb, nbc = outs
    return (loss.reshape(()), nnp.reshape(()), corr.reshape(()),
            nb.reshape(()), nbc.reshape(()))


# speedup vs baseline: 3.5401x; 3.5401x over previous
import math

import jax
import jax.numpy as jnp
from jax import lax
from jax.experimental import pallas as pl
from jax.experimental.pallas import tpu as pltpu
from jax.experimental.pallas import tpu_sc as plsc

_ABSTAIN_WEIGHT = 0.1
_NUM_WORKERS = 32
_LANES = 16


def _sc_body(nc, ns, piece_sizes,
             clp_hbm, idx_hbm, s2c_hbm,
             partmax_hbm, lpc_hbm,
             lmax_v, val_v, idx_v, gidx_v, gout_v,
             vals_s, idxs_s,
             sem0, sem1, sem):
    npiece = len(piece_sizes)
    spw = ns // _NUM_WORKERS
    cid = lax.axis_index("c")
    sid = lax.axis_index("s")
    wid = cid * _LANES + sid

    lane = lax.iota(jnp.int32, _LANES)
    ninf = jnp.full((_LANES,), -jnp.inf, jnp.float32)

    sems = (sem0, sem1)
    core_off = [16 * sum(piece_sizes[:p]) for p in range(npiece)]
    region = _LANES * max(piece_sizes)

    def _hbm_slice(p):
        return pl.ds(cid * (nc // 2) + core_off[p] + sid * piece_sizes[p],
                     piece_sizes[p])

    def _spmem_slice(b, p):
        return pl.ds(b * region + sid * piece_sizes[p], piece_sizes[p])

    def _issue(p):
        b = p % 2
        pltpu.async_copy(clp_hbm.at[_hbm_slice(p)],
                         vals_s.at[_spmem_slice(b, p)], sems[b])
        pltpu.async_copy(idx_hbm.at[_hbm_slice(p)],
                         idxs_s.at[_spmem_slice(b, p)], sems[b])

    def _wait(p):
        b = p % 2
        pltpu.make_async_copy(clp_hbm.at[_hbm_slice(p)],
                              vals_s.at[_spmem_slice(b, p)], sems[b]).wait()
        pltpu.make_async_copy(idx_hbm.at[_hbm_slice(p)],
                              idxs_s.at[_spmem_slice(b, p)], sems[b]).wait()

    _issue(0)
    if npiece > 1:
        _issue(1)

    @plsc.parallel_loop(0, (ns + _LANES) // _LANES, unroll=8)
    def _init(i):
        lmax_v[pl.ds(i * _LANES, _LANES)] = ninf

    priv = ns + lane

    for p in range(npiece):
        b = p % 2
        piece = piece_sizes[p]
        sub = piece // _LANES
        _wait(p)
        pltpu.sync_copy(
            vals_s.at[_spmem_slice(b, p)], val_v.at[pl.ds(0, piece)])
        pltpu.sync_copy(
            idxs_s.at[_spmem_slice(b, p)], idx_v.at[pl.ds(0, piece)])
        if p + 2 < npiece:
            _issue(p + 2)

        stride = sub + 1
        lane_base = lane * stride
        lane_end = jnp.minimum((lane + 1) * stride, piece)
        main_t = sub - (_LANES - 1)
        hs = plsc.load_gather(idx_v, [lane_base])

        def _rake(t, carry):
            m, cs, tgt = carry
            offs = lane_base + t
            v = plsc.load_gather(val_v, [offs])
            s = plsc.load_gather(idx_v, [offs])
            changed = s != cs
            plsc.store_scatter(lmax_v, [tgt], m, mask=changed)
            tgt = jnp.where(changed, s, tgt)
            m = jnp.where(changed, v, jnp.maximum(m, v))
            return m, s, tgt

        carry = plsc.parallel_loop(
            0, main_t, unroll=8, carry=(ninf, hs, priv))(_rake)

        def _rake_tail(t, carry):
            m, cs, tgt = carry
            offs_raw = lane_base + t
            act = offs_raw < lane_end
            offs = jnp.minimum(offs_raw, piece - 1)
            v = plsc.load_gather(val_v, [offs])
            s0 = plsc.load_gather(idx_v, [offs])
            s = jnp.where(act, s0, cs)
            changed = s != cs
            plsc.store_scatter(lmax_v, [tgt], m, mask=changed)
            tgt = jnp.where(changed, s, tgt)
            m = jnp.where(changed, v, jnp.where(act, jnp.maximum(m, v), m))
            return m, s, tgt

        m, cs, _ = plsc.parallel_loop(
            main_t, stride, unroll=4, carry=carry)(_rake_tail)

        headv = plsc.load_gather(lmax_v, [priv])
        for j in range(_LANES):
            g = plsc.load_gather(lmax_v, [hs])
            plsc.store_scatter(lmax_v, [hs], jnp.maximum(g, headv),
                               mask=lane == j)
        for j in range(_LANES):
            g = plsc.load_gather(lmax_v, [cs])
            plsc.store_scatter(lmax_v, [cs], jnp.maximum(g, m),
                               mask=lane == j)
        lmax_v[pl.ds(ns, _LANES)] = ninf

    pltpu.sync_copy(lmax_v.at[pl.ds(0, ns)], vals_s.at[pl.ds(sid * ns, ns)])
    out_copy = pltpu.async_copy(vals_s.at[pl.ds(sid * ns, ns)],
                                partmax_hbm.at[pl.ds(wid * ns, ns)], sem0)

    sbase = wid * spw
    pltpu.sync_copy(s2c_hbm.at[pl.ds(sbase, spw)], gidx_v)
    pltpu.async_copy(clp_hbm.at[gidx_v], gout_v, sem).wait()
    pltpu.sync_copy(gout_v, lpc_hbm.at[pl.ds(sbase, spw)])

    out_copy.wait()


def _tc_body(pm_ref, tail_ref, lpc_ref, bug_ref, np_ref,
             loss_ref, nnp_ref, corr_ref, nb_ref, nbc_ref):
    pm = pm_ref[...]
    tail = tail_ref[...]
    lpc = lpc_ref[...]
    bug = bug_ref[...]
    nonpad = np_ref[...]

    is_bug = bug == 1
    is_np = nonpad == 1

    seg_max = jnp.maximum(jnp.max(pm, axis=0), tail)
    lp = jnp.where(is_bug, lpc, tail)
    lp = jnp.where(is_np, lp, 0.0)
    lp = jnp.minimum(lp, math.log(0.995))
    lp = lp + jnp.where(jnp.logical_and(is_bug, is_np),
                        _ABSTAIN_WEIGHT * tail,
                        jnp.zeros_like(lp))

    nnp = jnp.sum(nonpad)
    correct = jnp.logical_and(seg_max == lp, is_np)
    nobug = jnp.logical_and(jnp.logical_not(is_bug), is_np)

    loss_ref[0, 0] = -jnp.sum(lp) / nnp.astype(jnp.float32)
    nnp_ref[0, 0] = nnp
    corr_ref[0, 0] = jnp.sum(correct.astype(jnp.int32))
    nb_ref[0, 0] = jnp.sum(nobug.astype(jnp.int32))
    nbc_ref[0, 0] = jnp.sum(jnp.logical_and(nobug, correct).astype(jnp.int32))


def kernel(candidate_log_probs, candidate_to_sample_idx, sample_has_bug,
           sample_to_correct_candidate_idx, sample_is_nonpad, train_step):
    nc = candidate_to_sample_idx.shape[0]
    ns = sample_has_bug.shape[0]
    chunk = nc // _NUM_WORKERS
    piece_sizes = [chunk // 4] * 4
    maxp = max(piece_sizes)
    region = _LANES * maxp

    sc_fn = pl.kernel(
        lambda *refs: _sc_body(nc, ns, piece_sizes, *refs),
        out_type=(
            jax.ShapeDtypeStruct((_NUM_WORKERS * ns,), jnp.float32),
            jax.ShapeDtypeStruct((ns,), jnp.float32),
        ),
        mesh=plsc.VectorSubcoreMesh(core_axis_name="c", subcore_axis_name="s"),
        compiler_params=pltpu.CompilerParams(needs_layout_passes=False),
        scratch_types=[
            pltpu.VMEM((ns + _LANES,), jnp.float32),
            pltpu.VMEM((maxp,), jnp.float32),
            pltpu.VMEM((maxp,), jnp.int32),
            pltpu.VMEM((ns // _NUM_WORKERS,), jnp.int32),
            pltpu.VMEM((ns // _NUM_WORKERS,), jnp.float32),
            pltpu.VMEM_SHARED((2 * region,), jnp.float32),
            pltpu.VMEM_SHARED((2 * region,), jnp.int32),
            pltpu.SemaphoreType.DMA,
            pltpu.SemaphoreType.DMA,
            pltpu.SemaphoreType.DMA,
        ],
    )
    partmax, lpc = sc_fn(candidate_log_probs, candidate_to_sample_idx,
                         sample_to_correct_candidate_idx)

    srows = ns // 128
    outs = pl.pallas_call(
        _tc_body,
        out_shape=(
            jax.ShapeDtypeStruct((1, 1), jnp.float32),
            jax.ShapeDtypeStruct((1, 1), jnp.int32),
            jax.ShapeDtypeStruct((1, 1), jnp.int32),
            jax.ShapeDtypeStruct((1, 1), jnp.int32),
            jax.ShapeDtypeStruct((1, 1), jnp.int32),
        ),
        out_specs=tuple(pl.BlockSpec(memory_space=pltpu.SMEM)
                        for _ in range(5)),
    )(
        partmax.reshape(_NUM_WORKERS, srows, 128),
        candidate_log_probs[nc:].reshape(srows, 128),
        lpc.reshape(srows, 128),
        sample_has_bug.astype(jnp.int32).reshape(srows, 128),
        sample_is_nonpad.astype(jnp.int32).reshape(srows, 128),
    )
    loss, nnp, corr, nb, nbc = outs
    return (loss.reshape(()), nnp.reshape(()), corr.reshape(()),
            nb.reshape(()), nbc.reshape(()))
